# baseline (speedup 1.0000x reference)
import functools

import jax
import jax.numpy as jnp
from jax import lax
from jax.experimental import pallas as pl
from jax.experimental.pallas import tpu as pltpu
from jax.experimental.pallas import tpu_sc as plsc

_CLASSES = 16
_COLS = 4
_FRAME = 38
_ROWT = 8
_COLT = 128
_ROWS = 48
_LANES = 16
_SC_BATCH = 16


def _block_origin(c, n):
    grid_num, frame_ratio = 512, 0.3
    frame_s = int(grid_num / _COLS * frame_ratio)
    g = grid_num
    row, col = c // _COLS, c % _COLS
    xc = int((n - g) // 2 + (col + 0.5) * (g // _COLS))
    yc = int((n - g) // 2 + (row + 0.5) * (g // _COLS))
    return yc - frame_s // 2, xc - frame_s // 2


def _geometry(n):
    origins = [_block_origin(c, n) for c in range(_CLASSES)]
    rskew = origins[0][0] % _ROWT
    cskew = origins[0][1] % _COLT
    assert all(y % _ROWT == rskew and x % _COLT == cskew for y, x in origins)
    assert rskew + _FRAME <= _ROWS
    grid_x0 = origins[0][1] - cskew
    spanw = _COLS * _COLT
    row_starts = [origins[gr * _COLS][0] - rskew for gr in range(_COLS)]
    return rskew, cskew, grid_x0, spanw, row_starts


def _sc_part(X, nb):
    B, H, W = X.shape
    rskew, cskew, grid_x0, spanw, row_starts = _geometry(W)

    info = plsc.get_sparse_core_info()
    NC, NS = info.num_cores, info.num_subcores
    NW = NC * NS

    mesh = plsc.VectorSubcoreMesh(core_axis_name="c", subcore_axis_name="s")

    @functools.partial(
        pl.kernel,
        out_type=jax.ShapeDtypeStruct((nb, _CLASSES), jnp.float32),
        mesh=mesh,
        compiler_params=pltpu.CompilerParams(needs_layout_passes=False),
        scratch_types=[
            pltpu.VMEM((_COLS, _ROWS, spanw), jnp.float32),
            pltpu.VMEM((_LANES,), jnp.float32),
            pltpu.SemaphoreType.DMA((_COLS,)),
        ],
    )
    def run(x_hbm, out_hbm, blocks_v, res_v, sems):
        wid = lax.axis_index("s") * NC + lax.axis_index("c")
        lane = lax.iota(jnp.int32, _LANES)
        tail_mask = lane < (cskew + _FRAME) - (cskew + 2 * _LANES)

        for b0 in range(0, nb, NW):
            nw_here = min(NW, nb - b0)
            b = b0 + wid

            @pl.when(wid < nw_here)
            def _():
                copies = []
                for gr in range(_COLS):
                    copies.append(pltpu.make_async_copy(
                        x_hbm.at[b, pl.ds(row_starts[gr], _ROWS),
                                 pl.ds(grid_x0, spanw)],
                        blocks_v.at[gr],
                        sems.at[gr],
                    ))
                for cp in copies:
                    cp.start()

                res = jnp.zeros((_LANES,), jnp.float32)
                zero = jnp.zeros((_LANES,), jnp.float32)
                for gr in range(_COLS):
                    copies[gr].wait()
                    for gc in range(_COLS):
                        cbase = gc * _COLT + cskew

                        def body(i, accs):
                            r = rskew + 2 * i
                            out = []
                            for dr in range(2):
                                for k in range(3):
                                    v = blocks_v[gr, r + dr,
                                                 pl.ds(cbase + k * _LANES,
                                                       _LANES)]
                                    out.append(accs[dr * 3 + k] + v * v)
                            return tuple(out)

                        accs = lax.fori_loop(0, _FRAME // 2, body, (zero,) * 6)
                        a0 = accs[0] + accs[3]
                        a1 = accs[1] + accs[4]
                        a2 = jnp.where(tail_mask, accs[2] + accs[5], 0.0)
                        s = jnp.sum(a0 + a1 + a2)
                        res = jnp.where(lane == gr * _COLS + gc, s, res)
                res_v[...] = res
                pltpu.sync_copy(res_v, out_hbm.at[b])

    return run(X)


def _tc_part(X, b_lo, nb):
    B, H, W = X.shape
    rskew, cskew, grid_x0, spanw, row_starts = _geometry(W)

    def body(x_any, out_ref, buf, sem):
        i = pl.program_id(0)
        b = i + b_lo
        copies = []
        for gr in range(_COLS):
            copies.append(pltpu.make_async_copy(
                x_any.at[b, pl.ds(row_starts[gr], _ROWS),
                         pl.ds(grid_x0, spanw)],
                buf.at[gr],
                sem,
            ))
        for cp in copies:
            cp.start()
        for cp in copies:
            cp.wait()
        sums = []
        for gr in range(_COLS):
            for gc in range(_COLS):
                v = buf[gr, rskew:rskew + _FRAME,
                        gc * _COLT + cskew:gc * _COLT + cskew + _FRAME]
                sums.append(jnp.sum(v * v))
        out_ref[pl.ds(i, 1), :] = jnp.stack(sums).reshape(1, _CLASSES)

    return pl.pallas_call(
        body,
        grid=(nb,),
        in_specs=[pl.BlockSpec(memory_space=pl.ANY)],
        out_specs=pl.BlockSpec((nb, _CLASSES), lambda i: (0, 0)),
        out_shape=jax.ShapeDtypeStruct((nb, _CLASSES), jnp.float32),
        scratch_shapes=[
            pltpu.VMEM((_COLS, _ROWS, spanw), jnp.float32),
            pltpu.SemaphoreType.DMA,
        ],
    )(X)


def kernel(X):
    B = X.shape[0]
    nb_sc = min(_SC_BATCH, B)
    y_sc = _sc_part(X, nb_sc)
    if nb_sc == B:
        return y_sc
    y_tc = _tc_part(X, nb_sc, B - nb_sc)
    return jnp.concatenate([y_sc, y_tc], axis=0)


# speedup vs baseline: 1.6577x; 1.6577x over previous
import functools

import jax
import jax.numpy as jnp
from jax import lax
from jax.experimental import pallas as pl
from jax.experimental.pallas import tpu as pltpu
from jax.experimental.pallas import tpu_sc as plsc

_CLASSES = 16
_COLS = 4
_FRAME = 38
_ROWT = 8
_COLT = 128
_ROWS = 48
_LANES = 16


def _block_origin(c, n):
    grid_num, frame_ratio = 512, 0.3
    frame_s = int(grid_num / _COLS * frame_ratio)
    g = grid_num
    row, col = c // _COLS, c % _COLS
    xc = int((n - g) // 2 + (col + 0.5) * (g // _COLS))
    yc = int((n - g) // 2 + (row + 0.5) * (g // _COLS))
    return yc - frame_s // 2, xc - frame_s // 2


def kernel(X):
    B, H, W = X.shape
    n = W
    origins = [_block_origin(c, n) for c in range(_CLASSES)]
    rskew = origins[0][0] % _ROWT
    cskew = origins[0][1] % _COLT
    assert all(y % _ROWT == rskew and x % _COLT == cskew for y, x in origins)
    assert rskew + _FRAME <= _ROWS

    info = plsc.get_sparse_core_info()
    NC, NS = info.num_cores, info.num_subcores
    NW = NC * NS

    grid_x0 = origins[0][1] - cskew
    spanw = _COLS * _COLT
    row_starts = [origins[gr * _COLS][0] - rskew for gr in range(_COLS)]

    mesh = plsc.VectorSubcoreMesh(core_axis_name="c", subcore_axis_name="s")

    @functools.partial(
        pl.kernel,
        out_type=jax.ShapeDtypeStruct((B, _CLASSES), jnp.float32),
        mesh=mesh,
        compiler_params=pltpu.CompilerParams(needs_layout_passes=False),
        scratch_types=[
            pltpu.VMEM((_CLASSES, _ROWS, _COLT), jnp.float32),
            pltpu.VMEM((_LANES,), jnp.float32),
            pltpu.SemaphoreType.DMA((_CLASSES,)),
        ],
    )
    def run(x_hbm, out_hbm, blocks_v, res_v, sems):
        wid = lax.axis_index("s") * NC + lax.axis_index("c")
        lane = lax.iota(jnp.int32, _LANES)

        tail_mask = lane < (_FRAME - 2 * _LANES)

        for b0 in range(0, B, NW):
            b = b0 + wid
            copies = []
            for c in range(_CLASSES):
                y0, x0 = origins[c]
                copies.append(pltpu.make_async_copy(
                    x_hbm.at[b, pl.ds(y0 - rskew, _ROWS),
                             pl.ds(x0 - cskew, _COLT)],
                    blocks_v.at[c],
                    sems.at[c],
                ))
            for cp in copies:
                cp.start()

            res = jnp.zeros((_LANES,), jnp.float32)
            zero = jnp.zeros((_LANES,), jnp.float32)
            for c in range(_CLASSES):
                copies[c].wait()

                def body(i, accs):
                    r = rskew + 2 * i
                    out = []
                    for dr in range(2):
                        for k in range(3):
                            v = blocks_v[c, r + dr,
                                         pl.ds(cskew + k * _LANES, _LANES)]
                            out.append(accs[dr * 3 + k] + v * v)
                    return tuple(out)

                accs = lax.fori_loop(0, _FRAME // 2, body, (zero,) * 6)
                a0 = accs[0] + accs[3]
                a1 = accs[1] + accs[4]
                a2 = jnp.where(tail_mask, accs[2] + accs[5], 0.0)
                s = jnp.sum(a0 + a1 + a2)
                res = jnp.where(lane == c, s, res)
            res_v[...] = res
            pltpu.sync_copy(res_v, out_hbm.at[b])

    return run(X)


# speedup vs baseline: 1.6625x; 1.0029x over previous
import functools

import jax
import jax.numpy as jnp
from jax import lax
from jax.experimental import pallas as pl
from jax.experimental.pallas import tpu as pltpu
from jax.experimental.pallas import tpu_sc as plsc

_CLASSES = 16
_COLS = 4
_FRAME = 38
_ROWT = 8
_COLT = 128
_ROWS = 48
_LANES = 16


def _block_origin(c, n):
    grid_num, frame_ratio = 512, 0.3
    frame_s = int(grid_num / _COLS * frame_ratio)
    g = grid_num
    row, col = c // _COLS, c % _COLS
    xc = int((n - g) // 2 + (col + 0.5) * (g // _COLS))
    yc = int((n - g) // 2 + (row + 0.5) * (g // _COLS))
    return yc - frame_s // 2, xc - frame_s // 2


def kernel(X):
    B, H, W = X.shape
    n = W
    origins = [_block_origin(c, n) for c in range(_CLASSES)]
    rskew = origins[0][0] % _ROWT
    cskew = origins[0][1] % _COLT
    assert all(y % _ROWT == rskew and x % _COLT == cskew for y, x in origins)
    assert rskew + _FRAME <= _ROWS

    info = plsc.get_sparse_core_info()
    NC, NS = info.num_cores, info.num_subcores
    NW = NC * NS

    mesh = plsc.VectorSubcoreMesh(core_axis_name="c", subcore_axis_name="s")

    @functools.partial(
        pl.kernel,
        out_type=jax.ShapeDtypeStruct((B, _CLASSES), jnp.float32),
        mesh=mesh,
        compiler_params=pltpu.CompilerParams(needs_layout_passes=False),
        scratch_types=[
            pltpu.VMEM((_CLASSES, _ROWS, _COLT), jnp.float32),
            pltpu.VMEM((_LANES,), jnp.float32),
            pltpu.SemaphoreType.DMA((_CLASSES,)),
        ],
    )
    def run(x_hbm, out_hbm, blocks_v, res_v, sems):
        wid = lax.axis_index("s") * NC + lax.axis_index("c")
        lane = lax.iota(jnp.int32, _LANES)

        tail_mask = lane < (_FRAME - 2 * _LANES)

        for b0 in range(0, B, NW):
            b = b0 + wid
            copies = []
            for c in range(_CLASSES):
                y0, x0 = origins[c]
                copies.append(pltpu.make_async_copy(
                    x_hbm.at[b, pl.ds(y0 - rskew, _ROWS),
                             pl.ds(x0 - cskew, _COLT)],
                    blocks_v.at[c],
                    sems.at[c],
                ))
            for cp in copies:
                cp.start()

            res = jnp.zeros((_LANES,), jnp.float32)
            zero = jnp.zeros((_LANES,), jnp.float32)
            for c in range(_CLASSES):
                copies[c].wait()

                def body(i, accs):
                    r = rskew + 2 * i
                    out = []
                    for dr in range(2):
                        for k in range(3):
                            v = blocks_v[c, r + dr,
                                         pl.ds(cskew + k * _LANES, _LANES)]
                            out.append(accs[dr * 3 + k] + v * v)
                    return tuple(out)

                accs = lax.fori_loop(0, _FRAME // 2, body, (zero,) * 6)
                a0 = accs[0] + accs[3]
                a1 = accs[1] + accs[4]
                a2 = jnp.where(tail_mask, accs[2] + accs[5], 0.0)
                s = jnp.sum(a0 + a1 + a2)
                res = jnp.where(lane == c, s, res)
            res_v[...] = res
            pltpu.sync_copy(res_v, out_hbm.at[b])

    return run(X)
